# Initial kernel scaffold; baseline (speedup 1.0000x reference)
import jax
import jax.numpy as jnp
from jax.experimental import pallas as pl


def kernel(input_ids, token_type_ids, input_mask, visual_embeds, visual_mask, tok_table, pos_table, seg_table, vpos_table, img_table, vseg_table, ln_g, ln_b, vln_g, vln_b):
    raise NotImplementedError("write your pallas kernel here")


# speedup vs baseline: 4.6721x; 4.6721x over previous
import jax
import jax.numpy as jnp
from jax.experimental import pallas as pl
from jax.experimental.pallas import tpu as pltpu
from jax.experimental.pallas import tpu_sc as plsc

_EPS = 1e-12
_GATHER_WINDOW = 128
_BB = 8


def _sc_gather(table, flat_ids):
    n = flat_ids.shape[0]
    e = table.shape[1]
    ids2 = flat_ids.reshape(1, n)
    mesh = plsc.VectorSubcoreMesh(core_axis_name="core",
                                  subcore_axis_name="subcore")

    @pl.kernel(out_type=jax.ShapeDtypeStruct((n, e), table.dtype), mesh=mesh)
    def gather_kernel(tab_hbm, idx_hbm, out_hbm):
        def body(idx_vmem, out_vmem):
            pltpu.sync_copy(tab_hbm.at[idx_vmem.at[0]], out_vmem)

        pltpu.emit_pipeline(
            body,
            grid=(n // _GATHER_WINDOW,),
            in_specs=[pl.BlockSpec((1, _GATHER_WINDOW), lambda i: (0, i))],
            out_specs=[pl.BlockSpec((_GATHER_WINDOW, e), lambda i: (i, 0))],
            core_axis_name=("core", "subcore"),
            dimension_semantics=(pltpu.PARALLEL,),
        )(idx_hbm, out_hbm)

    return gather_kernel(table, ids2)


def _tc_body(g_ref, tt_ref, ve_ref, pos_ref, seg_ref, vpos_ref, img_ref,
             vseg_ref, lng_ref, lnb_ref, vlng_ref, vlnb_ref, out_ref):
    f = ve_ref.shape[1]
    g = g_ref[...]
    tt = tt_ref[...].astype(jnp.float32)[..., None]
    seg0 = seg_ref[0]
    dseg = seg_ref[1] - seg_ref[0]
    text = g + pos_ref[...][None, :, :] + seg0[None, None, :] + tt * dseg[None, None, :]

    ve = ve_ref[...]
    mu = jnp.mean(ve, axis=-1, keepdims=True)
    var = jnp.mean((ve - mu) ** 2, axis=-1, keepdims=True)
    vn = (ve - mu) * jax.lax.rsqrt(var + _EPS) * vlng_ref[0] + vlnb_ref[0]
    v = vn + vpos_ref[1:1 + f][None, :, :] + vseg_ref[0][None, None, :]

    img_row = img_ref[0] + vpos_ref[0] + vseg_ref[0]
    img_b = jnp.broadcast_to(img_row[None, None, :], (g.shape[0], 1, g.shape[2]))

    full = jnp.concatenate([text, img_b, v], axis=1)
    mu2 = jnp.mean(full, axis=-1, keepdims=True)
    var2 = jnp.mean((full - mu2) ** 2, axis=-1, keepdims=True)
    out_ref[...] = (full - mu2) * jax.lax.rsqrt(var2 + _EPS) * lng_ref[0] + lnb_ref[0]


def kernel(input_ids, token_type_ids, input_mask, visual_embeds, visual_mask,
           tok_table, pos_table, seg_table, vpos_table, img_table, vseg_table,
           ln_g, ln_b, vln_g, vln_b):
    b, l = input_ids.shape
    f = visual_embeds.shape[1]
    e = tok_table.shape[1]
    s = l + f + 1

    g = _sc_gather(tok_table, input_ids.reshape(-1))
    g3 = g.reshape(b, l, e)
    pos_s = pos_table[:l]

    emb = pl.pallas_call(
        _tc_body,
        grid=(b // _BB,),
        in_specs=[
            pl.BlockSpec((_BB, l, e), lambda i: (i, 0, 0)),
            pl.BlockSpec((_BB, l), lambda i: (i, 0)),
            pl.BlockSpec((_BB, f, e), lambda i: (i, 0, 0)),
            pl.BlockSpec((l, e), lambda i: (0, 0)),
            pl.BlockSpec(seg_table.shape, lambda i: (0, 0)),
            pl.BlockSpec(vpos_table.shape, lambda i: (0, 0)),
            pl.BlockSpec((1, e), lambda i: (0, 0)),
            pl.BlockSpec((1, e), lambda i: (0, 0)),
            pl.BlockSpec((1, e), lambda i: (0, 0)),
            pl.BlockSpec((1, e), lambda i: (0, 0)),
            pl.BlockSpec((1, e), lambda i: (0, 0)),
            pl.BlockSpec((1, e), lambda i: (0, 0)),
        ],
        out_specs=pl.BlockSpec((_BB, s, e), lambda i: (i, 0, 0)),
        out_shape=jax.ShapeDtypeStruct((b, s, e), jnp.float32),
    )(g3, token_type_ids, visual_embeds, pos_s, seg_table, vpos_table,
      img_table, vseg_table, ln_g.reshape(1, e), ln_b.reshape(1, e),
      vln_g.reshape(1, e), vln_b.reshape(1, e))

    img_token_mask = (jnp.sum(visual_mask, axis=1, keepdims=True) > 0).astype(
        visual_mask.dtype)
    mask = jnp.concatenate([input_mask, img_token_mask, visual_mask], axis=1)
    return emb, mask


# speedup vs baseline: 4.6755x; 1.0007x over previous
import jax
import jax.numpy as jnp
from jax.experimental import pallas as pl
from jax.experimental.pallas import tpu as pltpu
from jax.experimental.pallas import tpu_sc as plsc

_EPS = 1e-12
_GATHER_WINDOW = 128
_BB = 8


def _sc_gather(table, flat_ids):
    n = flat_ids.shape[0]
    e = table.shape[1]
    ids2 = flat_ids.reshape(1, n)
    mesh = plsc.VectorSubcoreMesh(core_axis_name="core",
                                  subcore_axis_name="subcore")

    @pl.kernel(out_type=jax.ShapeDtypeStruct((n, e), table.dtype), mesh=mesh)
    def gather_kernel(tab_hbm, idx_hbm, out_hbm):
        def body(idx_vmem, out_vmem):
            pltpu.sync_copy(tab_hbm.at[idx_vmem.at[0]], out_vmem)

        pltpu.emit_pipeline(
            body,
            grid=(n // _GATHER_WINDOW,),
            in_specs=[pl.BlockSpec((1, _GATHER_WINDOW), lambda i: (0, i))],
            out_specs=[pl.BlockSpec((_GATHER_WINDOW, e), lambda i: (i, 0))],
            core_axis_name=("core", "subcore"),
            dimension_semantics=(pltpu.PARALLEL,),
        )(idx_hbm, out_hbm)

    return gather_kernel(table, ids2)


def _tc_body(g_ref, tt_ref, ve_ref, pos_ref, seg_ref, vpos_ref, img_ref,
             vseg_ref, lng_ref, lnb_ref, vlng_ref, vlnb_ref, out_ref):
    f = ve_ref.shape[1]
    g = g_ref[...]
    tt = tt_ref[...].astype(jnp.float32)[..., None]
    seg0 = seg_ref[0]
    dseg = seg_ref[1] - seg_ref[0]
    text = g + pos_ref[...][None, :, :] + seg0[None, None, :] + tt * dseg[None, None, :]

    ve = ve_ref[...]
    mu = jnp.mean(ve, axis=-1, keepdims=True)
    var = jnp.mean((ve - mu) ** 2, axis=-1, keepdims=True)
    vn = (ve - mu) * jax.lax.rsqrt(var + _EPS) * vlng_ref[0] + vlnb_ref[0]
    v = vn + vpos_ref[1:1 + f][None, :, :] + vseg_ref[0][None, None, :]

    img_row = img_ref[0] + vpos_ref[0] + vseg_ref[0]
    img_b = jnp.broadcast_to(img_row[None, None, :], (g.shape[0], 1, g.shape[2]))

    full = jnp.concatenate([text, img_b, v], axis=1)
    mu2 = jnp.mean(full, axis=-1, keepdims=True)
    var2 = jnp.mean((full - mu2) ** 2, axis=-1, keepdims=True)
    out_ref[...] = (full - mu2) * jax.lax.rsqrt(var2 + _EPS) * lng_ref[0] + lnb_ref[0]


def kernel(input_ids, token_type_ids, input_mask, visual_embeds, visual_mask,
           tok_table, pos_table, seg_table, vpos_table, img_table, vseg_table,
           ln_g, ln_b, vln_g, vln_b):
    b, l = input_ids.shape
    f = visual_embeds.shape[1]
    e = tok_table.shape[1]
    s = l + f + 1

    g = _sc_gather(tok_table, input_ids.reshape(-1))
    g3 = g.reshape(b, l, e)
    pos_s = pos_table[:l]

    emb = pl.pallas_call(
        _tc_body,
        grid=(b // _BB,),
        in_specs=[
            pl.BlockSpec((_BB, l, e), lambda i: (i, 0, 0)),
            pl.BlockSpec((_BB, l), lambda i: (i, 0)),
            pl.BlockSpec((_BB, f, e), lambda i: (i, 0, 0)),
            pl.BlockSpec((l, e), lambda i: (0, 0)),
            pl.BlockSpec(seg_table.shape, lambda i: (0, 0)),
            pl.BlockSpec(vpos_table.shape, lambda i: (0, 0)),
            pl.BlockSpec((1, e), lambda i: (0, 0)),
            pl.BlockSpec((1, e), lambda i: (0, 0)),
            pl.BlockSpec((1, e), lambda i: (0, 0)),
            pl.BlockSpec((1, e), lambda i: (0, 0)),
            pl.BlockSpec((1, e), lambda i: (0, 0)),
            pl.BlockSpec((1, e), lambda i: (0, 0)),
        ],
        out_specs=pl.BlockSpec((_BB, s, e), lambda i: (i, 0, 0)),
        out_shape=jax.ShapeDtypeStruct((b, s, e), jnp.float32),
        compiler_params=pltpu.CompilerParams(
            dimension_semantics=("parallel",)),
    )(g3, token_type_ids, visual_embeds, pos_s, seg_table, vpos_table,
      img_table, vseg_table, ln_g.reshape(1, e), ln_b.reshape(1, e),
      vln_g.reshape(1, e), vln_b.reshape(1, e))

    img_token_mask = (jnp.sum(visual_mask, axis=1, keepdims=True) > 0).astype(
        visual_mask.dtype)
    mask = jnp.concatenate([input_mask, img_token_mask, visual_mask], axis=1)
    return emb, mask


# speedup vs baseline: 5.5170x; 1.1800x over previous
import jax
import jax.numpy as jnp
from jax.experimental import pallas as pl
from jax.experimental.pallas import tpu as pltpu
from jax.experimental.pallas import tpu_sc as plsc

_EPS = 1e-12
_GATHER_WINDOW = 128
_BB = 32


def _sc_gather(table, flat_ids):
    n = flat_ids.shape[0]
    e = table.shape[1]
    ids2 = flat_ids.reshape(1, n)
    mesh = plsc.VectorSubcoreMesh(core_axis_name="core",
                                  subcore_axis_name="subcore")

    @pl.kernel(out_type=jax.ShapeDtypeStruct((n, e), table.dtype), mesh=mesh)
    def gather_kernel(tab_hbm, idx_hbm, out_hbm):
        def body(idx_vmem, out_vmem):
            pltpu.sync_copy(tab_hbm.at[idx_vmem.at[0]], out_vmem)

        pltpu.emit_pipeline(
            body,
            grid=(n // _GATHER_WINDOW,),
            in_specs=[pl.BlockSpec((1, _GATHER_WINDOW), lambda i: (0, i))],
            out_specs=[pl.BlockSpec((_GATHER_WINDOW, e), lambda i: (i, 0))],
            core_axis_name=("core", "subcore"),
            dimension_semantics=(pltpu.PARALLEL,),
        )(idx_hbm, out_hbm)

    return gather_kernel(table, ids2)


def _tc_body(g_ref, tt_ref, ve_ref, pos_ref, seg_ref, vpos_ref, img_ref,
             vseg_ref, lng_ref, lnb_ref, vlng_ref, vlnb_ref, out_ref):
    f = ve_ref.shape[1]
    g = g_ref[...]
    tt = tt_ref[...].astype(jnp.float32)[..., None]
    seg0 = seg_ref[0]
    dseg = seg_ref[1] - seg_ref[0]
    text = g + pos_ref[...][None, :, :] + seg0[None, None, :] + tt * dseg[None, None, :]

    ve = ve_ref[...]
    mu = jnp.mean(ve, axis=-1, keepdims=True)
    var = jnp.mean((ve - mu) ** 2, axis=-1, keepdims=True)
    vn = (ve - mu) * jax.lax.rsqrt(var + _EPS) * vlng_ref[0] + vlnb_ref[0]
    v = vn + vpos_ref[1:1 + f][None, :, :] + vseg_ref[0][None, None, :]

    img_row = img_ref[0] + vpos_ref[0] + vseg_ref[0]
    img_b = jnp.broadcast_to(img_row[None, None, :], (g.shape[0], 1, g.shape[2]))

    full = jnp.concatenate([text, img_b, v], axis=1)
    mu2 = jnp.mean(full, axis=-1, keepdims=True)
    var2 = jnp.mean((full - mu2) ** 2, axis=-1, keepdims=True)
    out_ref[...] = (full - mu2) * jax.lax.rsqrt(var2 + _EPS) * lng_ref[0] + lnb_ref[0]


def kernel(input_ids, token_type_ids, input_mask, visual_embeds, visual_mask,
           tok_table, pos_table, seg_table, vpos_table, img_table, vseg_table,
           ln_g, ln_b, vln_g, vln_b):
    b, l = input_ids.shape
    f = visual_embeds.shape[1]
    e = tok_table.shape[1]
    s = l + f + 1

    g = _sc_gather(tok_table, input_ids.reshape(-1))
    g3 = g.reshape(b, l, e)
    pos_s = pos_table[:l]

    emb = pl.pallas_call(
        _tc_body,
        grid=(b // _BB,),
        in_specs=[
            pl.BlockSpec((_BB, l, e), lambda i: (i, 0, 0)),
            pl.BlockSpec((_BB, l), lambda i: (i, 0)),
            pl.BlockSpec((_BB, f, e), lambda i: (i, 0, 0)),
            pl.BlockSpec((l, e), lambda i: (0, 0)),
            pl.BlockSpec(seg_table.shape, lambda i: (0, 0)),
            pl.BlockSpec(vpos_table.shape, lambda i: (0, 0)),
            pl.BlockSpec((1, e), lambda i: (0, 0)),
            pl.BlockSpec((1, e), lambda i: (0, 0)),
            pl.BlockSpec((1, e), lambda i: (0, 0)),
            pl.BlockSpec((1, e), lambda i: (0, 0)),
            pl.BlockSpec((1, e), lambda i: (0, 0)),
            pl.BlockSpec((1, e), lambda i: (0, 0)),
        ],
        out_specs=pl.BlockSpec((_BB, s, e), lambda i: (i, 0, 0)),
        out_shape=jax.ShapeDtypeStruct((b, s, e), jnp.float32),
        compiler_params=pltpu.CompilerParams(
            dimension_semantics=("parallel",)),
    )(g3, token_type_ids, visual_embeds, pos_s, seg_table, vpos_table,
      img_table, vseg_table, ln_g.reshape(1, e), ln_b.reshape(1, e),
      vln_g.reshape(1, e), vln_b.reshape(1, e))

    img_token_mask = (jnp.sum(visual_mask, axis=1, keepdims=True) > 0).astype(
        visual_mask.dtype)
    mask = jnp.concatenate([input_mask, img_token_mask, visual_mask], axis=1)
    return emb, mask


# speedup vs baseline: 5.5489x; 1.0058x over previous
import jax
import jax.numpy as jnp
from jax.experimental import pallas as pl
from jax.experimental.pallas import tpu as pltpu
from jax.experimental.pallas import tpu_sc as plsc

_EPS = 1e-12
_GATHER_WINDOW = 128
_BB = 64


def _sc_gather(table, flat_ids):
    n = flat_ids.shape[0]
    e = table.shape[1]
    ids2 = flat_ids.reshape(1, n)
    mesh = plsc.VectorSubcoreMesh(core_axis_name="core",
                                  subcore_axis_name="subcore")

    @pl.kernel(out_type=jax.ShapeDtypeStruct((n, e), table.dtype), mesh=mesh)
    def gather_kernel(tab_hbm, idx_hbm, out_hbm):
        def body(idx_vmem, out_vmem):
            pltpu.sync_copy(tab_hbm.at[idx_vmem.at[0]], out_vmem)

        pltpu.emit_pipeline(
            body,
            grid=(n // _GATHER_WINDOW,),
            in_specs=[pl.BlockSpec((1, _GATHER_WINDOW), lambda i: (0, i))],
            out_specs=[pl.BlockSpec((_GATHER_WINDOW, e), lambda i: (i, 0))],
            core_axis_name=("core", "subcore"),
            dimension_semantics=(pltpu.PARALLEL,),
        )(idx_hbm, out_hbm)

    return gather_kernel(table, ids2)


def _tc_body(g_ref, tt_ref, ve_ref, pos_ref, seg_ref, vpos_ref, img_ref,
             vseg_ref, lng_ref, lnb_ref, vlng_ref, vlnb_ref, out_ref):
    f = ve_ref.shape[1]
    g = g_ref[...]
    tt = tt_ref[...].astype(jnp.float32)[..., None]
    seg0 = seg_ref[0]
    dseg = seg_ref[1] - seg_ref[0]
    text = g + pos_ref[...][None, :, :] + seg0[None, None, :] + tt * dseg[None, None, :]

    ve = ve_ref[...]
    mu = jnp.mean(ve, axis=-1, keepdims=True)
    var = jnp.mean((ve - mu) ** 2, axis=-1, keepdims=True)
    vn = (ve - mu) * jax.lax.rsqrt(var + _EPS) * vlng_ref[0] + vlnb_ref[0]
    v = vn + vpos_ref[1:1 + f][None, :, :] + vseg_ref[0][None, None, :]

    img_row = img_ref[0] + vpos_ref[0] + vseg_ref[0]
    img_b = jnp.broadcast_to(img_row[None, None, :], (g.shape[0], 1, g.shape[2]))

    full = jnp.concatenate([text, img_b, v], axis=1)
    mu2 = jnp.mean(full, axis=-1, keepdims=True)
    var2 = jnp.mean((full - mu2) ** 2, axis=-1, keepdims=True)
    out_ref[...] = (full - mu2) * jax.lax.rsqrt(var2 + _EPS) * lng_ref[0] + lnb_ref[0]


def kernel(input_ids, token_type_ids, input_mask, visual_embeds, visual_mask,
           tok_table, pos_table, seg_table, vpos_table, img_table, vseg_table,
           ln_g, ln_b, vln_g, vln_b):
    b, l = input_ids.shape
    f = visual_embeds.shape[1]
    e = tok_table.shape[1]
    s = l + f + 1

    g = _sc_gather(tok_table, input_ids.reshape(-1))
    g3 = g.reshape(b, l, e)
    pos_s = pos_table[:l]

    emb = pl.pallas_call(
        _tc_body,
        grid=(b // _BB,),
        in_specs=[
            pl.BlockSpec((_BB, l, e), lambda i: (i, 0, 0)),
            pl.BlockSpec((_BB, l), lambda i: (i, 0)),
            pl.BlockSpec((_BB, f, e), lambda i: (i, 0, 0)),
            pl.BlockSpec((l, e), lambda i: (0, 0)),
            pl.BlockSpec(seg_table.shape, lambda i: (0, 0)),
            pl.BlockSpec(vpos_table.shape, lambda i: (0, 0)),
            pl.BlockSpec((1, e), lambda i: (0, 0)),
            pl.BlockSpec((1, e), lambda i: (0, 0)),
            pl.BlockSpec((1, e), lambda i: (0, 0)),
            pl.BlockSpec((1, e), lambda i: (0, 0)),
            pl.BlockSpec((1, e), lambda i: (0, 0)),
            pl.BlockSpec((1, e), lambda i: (0, 0)),
        ],
        out_specs=pl.BlockSpec((_BB, s, e), lambda i: (i, 0, 0)),
        out_shape=jax.ShapeDtypeStruct((b, s, e), jnp.float32),
        compiler_params=pltpu.CompilerParams(
            dimension_semantics=("parallel",)),
    )(g3, token_type_ids, visual_embeds, pos_s, seg_table, vpos_table,
      img_table, vseg_table, ln_g.reshape(1, e), ln_b.reshape(1, e),
      vln_g.reshape(1, e), vln_b.reshape(1, e))

    img_token_mask = (jnp.sum(visual_mask, axis=1, keepdims=True) > 0).astype(
        visual_mask.dtype)
    mask = jnp.concatenate([input_mask, img_token_mask, visual_mask], axis=1)
    return emb, mask


# speedup vs baseline: 6.9973x; 1.2610x over previous
import jax
import jax.numpy as jnp
from jax.experimental import pallas as pl
from jax.experimental.pallas import tpu as pltpu
from jax.experimental.pallas import tpu_sc as plsc

_EPS = 1e-12
_GATHER_WINDOW = 128
_BB = 64


def _sc_gather(table, flat_ids):
    n = flat_ids.shape[0]
    e = table.shape[1]
    ids2 = flat_ids.reshape(1, n)
    mesh = plsc.VectorSubcoreMesh(core_axis_name="core",
                                  subcore_axis_name="subcore")

    @pl.kernel(out_type=jax.ShapeDtypeStruct((n, e), table.dtype), mesh=mesh)
    def gather_kernel(tab_hbm, idx_hbm, out_hbm):
        def body(idx_vmem, out_vmem):
            pltpu.sync_copy(tab_hbm.at[idx_vmem.at[0]], out_vmem)

        pltpu.emit_pipeline(
            body,
            grid=(n // _GATHER_WINDOW,),
            in_specs=[pl.BlockSpec((1, _GATHER_WINDOW), lambda i: (0, i))],
            out_specs=[pl.BlockSpec((_GATHER_WINDOW, e), lambda i: (i, 0))],
            core_axis_name=("core", "subcore"),
            dimension_semantics=(pltpu.PARALLEL,),
        )(idx_hbm, out_hbm)

    return gather_kernel(table, ids2)


def _tc_body(g_ref, tt_ref, ve_ref, pos_ref, seg_ref, vpos_ref, img_ref,
             vseg_ref, lng_ref, lnb_ref, vlng_ref, vlnb_ref, out_ref):
    f = ve_ref.shape[0]
    bb = g_ref.shape[1]
    g = g_ref[...]
    tt = tt_ref[0].astype(jnp.float32)[..., None]
    seg0 = seg_ref[0]
    dseg = seg_ref[1] - seg_ref[0]
    text = g + pos_ref[...][:, None, :] + seg0[None, None, :] + tt * dseg[None, None, :]

    ve = ve_ref[...]
    mu = jnp.mean(ve, axis=-1, keepdims=True)
    var = jnp.mean((ve - mu) ** 2, axis=-1, keepdims=True)
    vn = (ve - mu) * jax.lax.rsqrt(var + _EPS) * vlng_ref[0] + vlnb_ref[0]
    v = vn + vpos_ref[1:1 + f][:, None, :] + vseg_ref[0][None, None, :]

    img_row = img_ref[0] + vpos_ref[0] + vseg_ref[0]
    img_b = jnp.broadcast_to(img_row[None, None, :], (1, bb, g.shape[2]))

    full = jnp.concatenate([text, img_b, v], axis=0)
    mu2 = jnp.mean(full, axis=-1, keepdims=True)
    var2 = jnp.mean((full - mu2) ** 2, axis=-1, keepdims=True)
    out_ref[...] = (full - mu2) * jax.lax.rsqrt(var2 + _EPS) * lng_ref[0] + lnb_ref[0]


def kernel(input_ids, token_type_ids, input_mask, visual_embeds, visual_mask,
           tok_table, pos_table, seg_table, vpos_table, img_table, vseg_table,
           ln_g, ln_b, vln_g, vln_b):
    b, l = input_ids.shape
    f = visual_embeds.shape[1]
    e = tok_table.shape[1]
    s = l + f + 1

    g = _sc_gather(tok_table, input_ids.T.reshape(-1))
    g3 = g.reshape(l, b, e)
    tt_t = token_type_ids.T.reshape(l, b // _BB, _BB).transpose(1, 0, 2)
    ve_t = visual_embeds.transpose(1, 0, 2)
    pos_s = pos_table[:l]

    emb_t = pl.pallas_call(
        _tc_body,
        grid=(b // _BB,),
        in_specs=[
            pl.BlockSpec((l, _BB, e), lambda i: (0, i, 0)),
            pl.BlockSpec((1, l, _BB), lambda i: (i, 0, 0)),
            pl.BlockSpec((f, _BB, e), lambda i: (0, i, 0)),
            pl.BlockSpec((l, e), lambda i: (0, 0)),
            pl.BlockSpec(seg_table.shape, lambda i: (0, 0)),
            pl.BlockSpec(vpos_table.shape, lambda i: (0, 0)),
            pl.BlockSpec((1, e), lambda i: (0, 0)),
            pl.BlockSpec((1, e), lambda i: (0, 0)),
            pl.BlockSpec((1, e), lambda i: (0, 0)),
            pl.BlockSpec((1, e), lambda i: (0, 0)),
            pl.BlockSpec((1, e), lambda i: (0, 0)),
            pl.BlockSpec((1, e), lambda i: (0, 0)),
        ],
        out_specs=pl.BlockSpec((s, _BB, e), lambda i: (0, i, 0)),
        out_shape=jax.ShapeDtypeStruct((s, b, e), jnp.float32),
        compiler_params=pltpu.CompilerParams(
            dimension_semantics=("parallel",)),
    )(g3, tt_t, ve_t, pos_s, seg_table, vpos_table,
      img_table, vseg_table, ln_g.reshape(1, e), ln_b.reshape(1, e),
      vln_g.reshape(1, e), vln_b.reshape(1, e))
    emb = emb_t.transpose(1, 0, 2)

    img_token_mask = (jnp.sum(visual_mask, axis=1, keepdims=True) > 0).astype(
        visual_mask.dtype)
    mask = jnp.concatenate([input_mask, img_token_mask, visual_mask], axis=1)
    return emb, mask


# speedup vs baseline: 7.5513x; 1.0792x over previous
import jax
import jax.numpy as jnp
from jax.experimental import pallas as pl
from jax.experimental.pallas import tpu as pltpu
from jax.experimental.pallas import tpu_sc as plsc

_EPS = 1e-12
_GATHER_WINDOW = 128
_BB = 64
_NC = 4


def _sc_gather(table, flat_ids):
    n = flat_ids.shape[0]
    e = table.shape[1]
    ids2 = flat_ids.reshape(1, n)
    mesh = plsc.VectorSubcoreMesh(core_axis_name="core",
                                  subcore_axis_name="subcore")

    @pl.kernel(out_type=jax.ShapeDtypeStruct((n, e), table.dtype), mesh=mesh)
    def gather_kernel(tab_hbm, idx_hbm, out_hbm):
        def body(idx_vmem, out_vmem):
            pltpu.sync_copy(tab_hbm.at[idx_vmem.at[0]], out_vmem)

        pltpu.emit_pipeline(
            body,
            grid=(n // _GATHER_WINDOW,),
            in_specs=[pl.BlockSpec((1, _GATHER_WINDOW), lambda i: (0, i))],
            out_specs=[pl.BlockSpec((_GATHER_WINDOW, e), lambda i: (i, 0))],
            core_axis_name=("core", "subcore"),
            dimension_semantics=(pltpu.PARALLEL,),
        )(idx_hbm, out_hbm)

    return gather_kernel(table, ids2)


def _tc_body(*refs):
    (g_ref, tt_ref, ve_ref, pos_ref, seg_ref, vpos_ref, img_ref,
     vseg_ref, lng_ref, lnb_ref, vlng_ref, vlnb_ref) = refs[:12]
    out_ref = refs[-1]
    f = ve_ref.shape[0]
    bb = g_ref.shape[1]
    g = g_ref[...]
    tt = tt_ref[0].astype(jnp.float32)[..., None]
    seg0 = seg_ref[0]
    dseg = seg_ref[1] - seg_ref[0]
    text = g + pos_ref[...][:, None, :] + seg0[None, None, :] + tt * dseg[None, None, :]

    ve = ve_ref[...]
    mu = jnp.mean(ve, axis=-1, keepdims=True)
    var = jnp.mean((ve - mu) ** 2, axis=-1, keepdims=True)
    vn = (ve - mu) * jax.lax.rsqrt(var + _EPS) * vlng_ref[0] + vlnb_ref[0]
    v = vn + vpos_ref[1:1 + f][:, None, :] + vseg_ref[0][None, None, :]

    img_row = img_ref[0] + vpos_ref[0] + vseg_ref[0]
    img_b = jnp.broadcast_to(img_row[None, None, :], (1, bb, g.shape[2]))

    full = jnp.concatenate([text, img_b, v], axis=0)
    mu2 = jnp.mean(full, axis=-1, keepdims=True)
    var2 = jnp.mean((full - mu2) ** 2, axis=-1, keepdims=True)
    out_ref[...] = (full - mu2) * jax.lax.rsqrt(var2 + _EPS) * lng_ref[0] + lnb_ref[0]


def kernel(input_ids, token_type_ids, input_mask, visual_embeds, visual_mask,
           tok_table, pos_table, seg_table, vpos_table, img_table, vseg_table,
           ln_g, ln_b, vln_g, vln_b):
    b, l = input_ids.shape
    f = visual_embeds.shape[1]
    e = tok_table.shape[1]
    s = l + f + 1

    pos_s = pos_table[:l]
    consts = (pos_s, seg_table, vpos_table, img_table, vseg_table,
              ln_g.reshape(1, e), ln_b.reshape(1, e),
              vln_g.reshape(1, e), vln_b.reshape(1, e))
    const_specs = [
        pl.BlockSpec((l, e), lambda i: (0, 0)),
        pl.BlockSpec(seg_table.shape, lambda i: (0, 0)),
        pl.BlockSpec(vpos_table.shape, lambda i: (0, 0)),
        pl.BlockSpec((1, e), lambda i: (0, 0)),
        pl.BlockSpec((1, e), lambda i: (0, 0)),
        pl.BlockSpec((1, e), lambda i: (0, 0)),
        pl.BlockSpec((1, e), lambda i: (0, 0)),
        pl.BlockSpec((1, e), lambda i: (0, 0)),
        pl.BlockSpec((1, e), lambda i: (0, 0)),
    ]

    bc = b // _NC
    steps = bc // _BB
    emb_t = None
    for c in range(_NC):
        sl = slice(c * bc, (c + 1) * bc)
        g3 = _sc_gather(tok_table, input_ids[sl].T.reshape(-1)).reshape(l, bc, e)
        tt_t = (token_type_ids[sl].T.reshape(l, steps, _BB)
                .transpose(1, 0, 2))
        ve_t = visual_embeds[sl].transpose(1, 0, 2)
        in_specs = [
            pl.BlockSpec((l, _BB, e), lambda i: (0, i, 0)),
            pl.BlockSpec((1, l, _BB), lambda i: (i, 0, 0)),
            pl.BlockSpec((f, _BB, e), lambda i: (0, i, 0)),
        ] + list(const_specs)
        args = (g3, tt_t, ve_t) + consts
        off = c * steps
        if emb_t is None:
            emb_t = pl.pallas_call(
                _tc_body,
                grid=(steps,),
                in_specs=in_specs,
                out_specs=pl.BlockSpec((s, _BB, e),
                                       lambda i, off=off: (0, off + i, 0)),
                out_shape=jax.ShapeDtypeStruct((s, b, e), jnp.float32),
                compiler_params=pltpu.CompilerParams(
                    dimension_semantics=("arbitrary",)),
            )(*args)
        else:
            emb_t = pl.pallas_call(
                _tc_body,
                grid=(steps,),
                in_specs=in_specs + [
                    pl.BlockSpec(memory_space=pltpu.MemorySpace.HBM)],
                out_specs=pl.BlockSpec((s, _BB, e),
                                       lambda i, off=off: (0, off + i, 0)),
                out_shape=jax.ShapeDtypeStruct((s, b, e), jnp.float32),
                input_output_aliases={12: 0},
                compiler_params=pltpu.CompilerParams(
                    dimension_semantics=("arbitrary",)),
            )(*args, emb_t)
    emb = emb_t.transpose(1, 0, 2)

    img_token_mask = (jnp.sum(visual_mask, axis=1, keepdims=True) > 0).astype(
        visual_mask.dtype)
    mask = jnp.concatenate([input_mask, img_token_mask, visual_mask], axis=1)
    return emb, mask


# speedup vs baseline: 7.7230x; 1.0227x over previous
import jax
import jax.numpy as jnp
from jax.experimental import pallas as pl
from jax.experimental.pallas import tpu as pltpu
from jax.experimental.pallas import tpu_sc as plsc

_EPS = 1e-12
_GATHER_WINDOW = 128
_BB = 32
_NC = 4


def _sc_gather(table, flat_ids):
    n = flat_ids.shape[0]
    e = table.shape[1]
    ids2 = flat_ids.reshape(1, n)
    mesh = plsc.VectorSubcoreMesh(core_axis_name="core",
                                  subcore_axis_name="subcore")

    @pl.kernel(out_type=jax.ShapeDtypeStruct((n, e), table.dtype), mesh=mesh)
    def gather_kernel(tab_hbm, idx_hbm, out_hbm):
        def body(idx_vmem, out_vmem):
            pltpu.sync_copy(tab_hbm.at[idx_vmem.at[0]], out_vmem)

        pltpu.emit_pipeline(
            body,
            grid=(n // _GATHER_WINDOW,),
            in_specs=[pl.BlockSpec((1, _GATHER_WINDOW), lambda i: (0, i))],
            out_specs=[pl.BlockSpec((_GATHER_WINDOW, e), lambda i: (i, 0))],
            core_axis_name=("core", "subcore"),
            dimension_semantics=(pltpu.PARALLEL,),
        )(idx_hbm, out_hbm)

    return gather_kernel(table, ids2)


def _tc_body(*refs):
    (g_ref, tt_ref, ve_ref, pos_ref, seg_ref, vpos_ref, img_ref,
     vseg_ref, lng_ref, lnb_ref, vlng_ref, vlnb_ref) = refs[:12]
    out_ref = refs[-1]
    f = ve_ref.shape[0]
    bb = g_ref.shape[1]
    g = g_ref[...]
    tt = tt_ref[0].astype(jnp.float32)[..., None]
    seg0 = seg_ref[0]
    dseg = seg_ref[1] - seg_ref[0]
    text = g + pos_ref[...][:, None, :] + seg0[None, None, :] + tt * dseg[None, None, :]

    ve = ve_ref[...]
    mu = jnp.mean(ve, axis=-1, keepdims=True)
    var = jnp.mean((ve - mu) ** 2, axis=-1, keepdims=True)
    vn = (ve - mu) * jax.lax.rsqrt(var + _EPS) * vlng_ref[0] + vlnb_ref[0]
    v = vn + vpos_ref[1:1 + f][:, None, :] + vseg_ref[0][None, None, :]

    img_row = img_ref[0] + vpos_ref[0] + vseg_ref[0]
    img_b = jnp.broadcast_to(img_row[None, None, :], (1, bb, g.shape[2]))

    full = jnp.concatenate([text, img_b, v], axis=0)
    mu2 = jnp.mean(full, axis=-1, keepdims=True)
    var2 = jnp.mean((full - mu2) ** 2, axis=-1, keepdims=True)
    out_ref[...] = (full - mu2) * jax.lax.rsqrt(var2 + _EPS) * lng_ref[0] + lnb_ref[0]


def kernel(input_ids, token_type_ids, input_mask, visual_embeds, visual_mask,
           tok_table, pos_table, seg_table, vpos_table, img_table, vseg_table,
           ln_g, ln_b, vln_g, vln_b):
    b, l = input_ids.shape
    f = visual_embeds.shape[1]
    e = tok_table.shape[1]
    s = l + f + 1

    pos_s = pos_table[:l]
    consts = (pos_s, seg_table, vpos_table, img_table, vseg_table,
              ln_g.reshape(1, e), ln_b.reshape(1, e),
              vln_g.reshape(1, e), vln_b.reshape(1, e))
    const_specs = [
        pl.BlockSpec((l, e), lambda i: (0, 0)),
        pl.BlockSpec(seg_table.shape, lambda i: (0, 0)),
        pl.BlockSpec(vpos_table.shape, lambda i: (0, 0)),
        pl.BlockSpec((1, e), lambda i: (0, 0)),
        pl.BlockSpec((1, e), lambda i: (0, 0)),
        pl.BlockSpec((1, e), lambda i: (0, 0)),
        pl.BlockSpec((1, e), lambda i: (0, 0)),
        pl.BlockSpec((1, e), lambda i: (0, 0)),
        pl.BlockSpec((1, e), lambda i: (0, 0)),
    ]

    bc = b // _NC
    steps = bc // _BB
    tt_t = token_type_ids.T.reshape(l, b // _BB, _BB).transpose(1, 0, 2)
    ve_t = visual_embeds.transpose(1, 0, 2)
    emb_t = None
    for c in range(_NC):
        sl = slice(c * bc, (c + 1) * bc)
        g3 = _sc_gather(tok_table, input_ids[sl].T.reshape(-1)).reshape(l, bc, e)
        off = c * steps
        in_specs = [
            pl.BlockSpec((l, _BB, e), lambda i: (0, i, 0)),
            pl.BlockSpec((1, l, _BB), lambda i, off=off: (off + i, 0, 0)),
            pl.BlockSpec((f, _BB, e), lambda i, off=off: (0, off + i, 0)),
        ] + list(const_specs)
        args = (g3, tt_t, ve_t) + consts
        if emb_t is None:
            emb_t = pl.pallas_call(
                _tc_body,
                grid=(steps,),
                in_specs=in_specs,
                out_specs=pl.BlockSpec((s, _BB, e),
                                       lambda i, off=off: (0, off + i, 0)),
                out_shape=jax.ShapeDtypeStruct((s, b, e), jnp.float32),
                compiler_params=pltpu.CompilerParams(
                    dimension_semantics=("arbitrary",)),
            )(*args)
        else:
            emb_t = pl.pallas_call(
                _tc_body,
                grid=(steps,),
                in_specs=in_specs + [
                    pl.BlockSpec(memory_space=pltpu.MemorySpace.HBM)],
                out_specs=pl.BlockSpec((s, _BB, e),
                                       lambda i, off=off: (0, off + i, 0)),
                out_shape=jax.ShapeDtypeStruct((s, b, e), jnp.float32),
                input_output_aliases={12: 0},
                compiler_params=pltpu.CompilerParams(
                    dimension_semantics=("arbitrary",)),
            )(*args, emb_t)
    emb = emb_t.transpose(1, 0, 2)

    img_token_mask = (jnp.sum(visual_mask, axis=1, keepdims=True) > 0).astype(
        visual_mask.dtype)
    mask = jnp.concatenate([input_mask, img_token_mask, visual_mask], axis=1)
    return emb, mask
